# baseline (device time: 32868 ns/iter reference)
import jax
import jax.numpy as jnp
from jax import lax
from jax.experimental import pallas as pl
from jax.experimental.pallas import tpu as pltpu

N_DEV = 4
WINDOW = 128
NEG = -1e9
CHUNK_ROWS = (256, 128)
NCHUNK = len(CHUNK_ROWS)

_DevId = getattr(pl, "DeviceIdType", None) or pltpu.DeviceIdType
_sem_signal = getattr(pl, "semaphore_signal", None) or pltpu.semaphore_signal
_sem_wait = getattr(pl, "semaphore_wait", None) or pltpu.semaphore_wait


def kernel(x, Wq, K_ext, V_ext, Wo):
    B, Sq, D = x.shape
    Dh = 64
    H = Wq.shape[1] // Dh
    HD = H * Dh
    SqQ = Sq // N_DEV
    SqH = Sq - CHUNK_ROWS[1]

    K2 = K_ext.reshape(B, K_ext.shape[1], -1)
    V2 = V_ext.reshape(B, V_ext.shape[1], -1)

    def body(x_ref, wq_ref, k_ref, v_ref, wo_ref, out_ref,
             kvsend0, kvsend1, kvrecv0, kvrecv1,
             pbuf, rsrecv, agsend, agrecv,
             kvsend_sems, kvrecv_sems,
             rssend_sems, rsrecv_sems, agsend_sems, agrecv_sems):
        my = lax.axis_index("i")

        bar = pltpu.get_barrier_semaphore()
        for k in range(1, N_DEV):
            _sem_signal(bar, inc=1, device_id=((my + k) % N_DEV,),
                        device_id_type=_DevId.MESH)
        _sem_wait(bar, N_DEV - 1)

        for s, (kvs, kvr, rows) in enumerate(
                ((kvsend0, kvrecv0, CHUNK_ROWS[0]),
                 (kvsend1, kvrecv1, CHUNK_ROWS[1]))):
            @pl.when(my == s)
            def _(s=s, kvs=kvs, kvr=kvr, rows=rows):
                dsts = [(s + 2) % N_DEV, (s + 1) % N_DEV, (s + 3) % N_DEV]
                for cnt, j in enumerate(dsts):
                    kvs[cnt, 0] = k_ref[:, :rows,
                                        j * HD:(j + 1) * HD].astype(jnp.bfloat16)
                    kvs[cnt, 1] = v_ref[:, :rows,
                                        j * HD:(j + 1) * HD].astype(jnp.bfloat16)
                    pltpu.make_async_remote_copy(
                        src_ref=kvs.at[cnt], dst_ref=kvr,
                        send_sem=kvsend_sems.at[s, cnt], recv_sem=kvrecv_sems.at[s],
                        device_id=(j,), device_id_type=_DevId.MESH,
                    ).start()
                kvr[0] = k_ref[:, :rows, s * HD:(s + 1) * HD].astype(jnp.bfloat16)
                kvr[1] = v_ref[:, :rows, s * HD:(s + 1) * HD].astype(jnp.bfloat16)

        Q = []
        for b in range(B):
            Q.append((lax.dot_general(
                x_ref[b].astype(jnp.bfloat16), wq_ref[:, :].astype(jnp.bfloat16),
                (((1,), (0,)), ((), ())), preferred_element_type=jnp.float32)
                * (0.125 * 1.4426950408889634)).astype(jnp.bfloat16))

        def bias(nrow, row0, ncol, col0):
            r = lax.broadcasted_iota(jnp.int32, (nrow, ncol), 0) + row0
            c = lax.broadcasted_iota(jnp.int32, (nrow, ncol), 1) + col0
            return jnp.where(jnp.abs(r - c) <= WINDOW, 0.0, NEG).astype(jnp.float32)

        bias0 = bias(Sq, 0, CHUNK_ROWS[0], 0)
        bias1 = bias(CHUNK_ROWS[1], SqH, CHUNK_ROWS[1], CHUNK_ROWS[0])

        def wait_kv(c, kvr):
            @pl.when(my != c)
            def _():
                pltpu.make_async_remote_copy(
                    src_ref=kvr, dst_ref=kvr,
                    send_sem=kvsend_sems.at[c, 0], recv_sem=kvrecv_sems.at[c],
                    device_id=(0,), device_id_type=_DevId.MESH,
                ).wait_recv()

        wait_kv(0, kvrecv0)
        e0s = [[None] * H for _ in range(B)]
        l0 = [[None] * H for _ in range(B)]
        c0 = [[None] * H for _ in range(B)]
        for b in range(B):
            for h in range(H):
                q = Q[b][:, h * Dh:(h + 1) * Dh]
                e0 = jnp.exp2(lax.dot_general(
                    q, kvrecv0[0, b, :, h * Dh:(h + 1) * Dh],
                    (((1,), (1,)), ((), ())),
                    preferred_element_type=jnp.float32) + bias0)
                l0[b][h] = jnp.sum(e0, axis=1, keepdims=True)
                e0s[b][h] = e0.astype(jnp.bfloat16)
        for b in range(B):
            for h in range(H):
                c0[b][h] = lax.dot_general(
                    e0s[b][h], kvrecv0[1, b, :, h * Dh:(h + 1) * Dh],
                    (((1,), (0,)), ((), ())), preferred_element_type=jnp.float32)

        wait_kv(1, kvrecv1)
        for b in range(B):
            acc = jnp.zeros((Sq, D), jnp.float32)
            for h in range(H):
                q_hi = Q[b][SqH:, h * Dh:(h + 1) * Dh]
                e1 = jnp.exp2(lax.dot_general(
                    q_hi, kvrecv1[0, b, :, h * Dh:(h + 1) * Dh],
                    (((1,), (1,)), ((), ())),
                    preferred_element_type=jnp.float32) + bias1)
                ctx_hi = c0[b][h][SqH:, :] + lax.dot_general(
                    e1.astype(jnp.bfloat16), kvrecv1[1, b, :, h * Dh:(h + 1) * Dh],
                    (((1,), (0,)), ((), ())), preferred_element_type=jnp.float32)
                l_hi = l0[b][h][SqH:, :] + jnp.sum(e1, axis=1, keepdims=True)
                ctx = jnp.concatenate(
                    [c0[b][h][:SqH, :] * (1.0 / l0[b][h][:SqH, :]),
                     ctx_hi * (1.0 / l_hi)], axis=0)
                acc = acc + lax.dot_general(
                    ctx.astype(jnp.bfloat16),
                    wo_ref[h * Dh:(h + 1) * Dh, :].astype(jnp.bfloat16),
                    (((1,), (0,)), ((), ())), preferred_element_type=jnp.float32)
            pbuf[b] = acc.astype(jnp.bfloat16)

        for k in (2, 1, 3):
            d = (my + k) % N_DEV
            pltpu.make_async_remote_copy(
                src_ref=pbuf.at[:, pl.ds(d * SqQ, SqQ), :],
                dst_ref=rsrecv.at[N_DEV - 1 - k],
                send_sem=rssend_sems.at[k - 1],
                recv_sem=rsrecv_sems.at[N_DEV - 1 - k],
                device_id=(d,), device_id_type=_DevId.MESH,
            ).start()

        for s in range(NCHUNK):
            @pl.when(my == s)
            def _(s=s):
                kvr = kvrecv0 if s == 0 else kvrecv1
                kvs = kvsend0 if s == 0 else kvsend1
                for cnt in range(N_DEV - 1):
                    pltpu.make_async_remote_copy(
                        src_ref=kvs.at[cnt], dst_ref=kvr,
                        send_sem=kvsend_sems.at[s, cnt], recv_sem=kvrecv_sems.at[s],
                        device_id=(0,), device_id_type=_DevId.MESH,
                    ).wait_send()

        for j in range(N_DEV - 1):
            pltpu.make_async_remote_copy(
                src_ref=rsrecv.at[j], dst_ref=rsrecv.at[j],
                send_sem=rssend_sems.at[0], recv_sem=rsrecv_sems.at[j],
                device_id=(0,), device_id_type=_DevId.MESH,
            ).wait_recv()
        sums = []
        for b in range(B):
            sum_q = pbuf[b, pl.ds(my * SqQ, SqQ), :].astype(jnp.float32)
            for j in range(N_DEV - 1):
                sum_q = sum_q + rsrecv[j, b].astype(jnp.float32)
            sums.append(sum_q)
            agsend[b] = sum_q.astype(jnp.bfloat16)
        for k in (2, 1, 3):
            pltpu.make_async_remote_copy(
                src_ref=agsend, dst_ref=agrecv.at[N_DEV - 1 - k],
                send_sem=agsend_sems.at[k - 1],
                recv_sem=agrecv_sems.at[N_DEV - 1 - k],
                device_id=((my + k) % N_DEV,), device_id_type=_DevId.MESH,
            ).start()

        for b in range(B):
            out_ref[b, pl.ds(my * SqQ, SqQ), :] = sums[b]
        for j in range(N_DEV - 1):
            pltpu.make_async_remote_copy(
                src_ref=agsend, dst_ref=agrecv.at[j],
                send_sem=agsend_sems.at[0], recv_sem=agrecv_sems.at[j],
                device_id=(0,), device_id_type=_DevId.MESH,
            ).wait_recv()
            src = (my + j + 1) % N_DEV
            for b in range(B):
                out_ref[b, pl.ds(src * SqQ, SqQ), :] = \
                    agrecv[j, b].astype(jnp.float32)

        for k in range(1, N_DEV):
            pltpu.make_async_remote_copy(
                src_ref=pbuf.at[:, pl.ds(0, SqQ), :], dst_ref=rsrecv.at[0],
                send_sem=rssend_sems.at[k - 1], recv_sem=rsrecv_sems.at[0],
                device_id=(0,), device_id_type=_DevId.MESH,
            ).wait_send()
            pltpu.make_async_remote_copy(
                src_ref=agsend, dst_ref=agrecv.at[0],
                send_sem=agsend_sems.at[k - 1], recv_sem=agrecv_sems.at[0],
                device_id=(0,), device_id_type=_DevId.MESH,
            ).wait_send()

    return pl.pallas_call(
        body,
        out_shape=jax.ShapeDtypeStruct((B, Sq, D), jnp.float32),
        in_specs=[pl.BlockSpec(memory_space=pltpu.VMEM)] * 5,
        out_specs=pl.BlockSpec(memory_space=pltpu.VMEM),
        scratch_shapes=[
            pltpu.VMEM((N_DEV - 1, 2, B, CHUNK_ROWS[0], HD), jnp.bfloat16),
            pltpu.VMEM((N_DEV - 1, 2, B, CHUNK_ROWS[1], HD), jnp.bfloat16),
            pltpu.VMEM((2, B, CHUNK_ROWS[0], HD), jnp.bfloat16),
            pltpu.VMEM((2, B, CHUNK_ROWS[1], HD), jnp.bfloat16),
            pltpu.VMEM((B, Sq, D), jnp.bfloat16),
            pltpu.VMEM((N_DEV - 1, B, Sq // N_DEV, D), jnp.bfloat16),
            pltpu.VMEM((B, Sq // N_DEV, D), jnp.bfloat16),
            pltpu.VMEM((N_DEV - 1, B, Sq // N_DEV, D), jnp.bfloat16),
            pltpu.SemaphoreType.DMA((NCHUNK, N_DEV - 1)),
            pltpu.SemaphoreType.DMA((NCHUNK,)),
            pltpu.SemaphoreType.DMA((N_DEV - 1,)),
            pltpu.SemaphoreType.DMA((N_DEV - 1,)),
            pltpu.SemaphoreType.DMA((N_DEV - 1,)),
            pltpu.SemaphoreType.DMA((N_DEV - 1,)),
        ],
        compiler_params=pltpu.CompilerParams(collective_id=0),
    )(x, Wq, K2, V2, Wo)


# device time: 30840 ns/iter; 1.0658x vs baseline; 1.0658x over previous
import jax
import jax.numpy as jnp
from jax import lax
from jax.experimental import pallas as pl
from jax.experimental.pallas import tpu as pltpu

N_DEV = 4
WINDOW = 128
NEG = -1e9
CHUNK_ROWS = (256, 128)
NCHUNK = len(CHUNK_ROWS)

_DevId = getattr(pl, "DeviceIdType", None) or pltpu.DeviceIdType
_sem_signal = getattr(pl, "semaphore_signal", None) or pltpu.semaphore_signal
_sem_wait = getattr(pl, "semaphore_wait", None) or pltpu.semaphore_wait


def kernel(x, Wq, K_ext, V_ext, Wo):
    B, Sq, D = x.shape
    Dh = 64
    H = Wq.shape[1] // Dh
    HD = H * Dh
    SqQ = Sq // N_DEV

    K2 = K_ext.reshape(B, K_ext.shape[1], -1)
    V2 = V_ext.reshape(B, V_ext.shape[1], -1)

    def body(x_ref, wq_ref, k_ref, v_ref, wo_ref, out_ref,
             ksend0, vsend0, ksend1, vsend1, krecv0, vrecv0, krecv1, vrecv1,
             pbuf, rsrecv, agsend, agrecv,
             ksend_sems, vsend_sems, krecv_sems, vrecv_sems,
             rssend_sems, rsrecv_sems, agsend_sems, agrecv_sems):
        my = lax.axis_index("i")

        bar = pltpu.get_barrier_semaphore()
        for k in range(1, N_DEV):
            _sem_signal(bar, inc=1, device_id=((my + k) % N_DEV,),
                        device_id_type=_DevId.MESH)
        _sem_wait(bar, N_DEV - 1)

        for s, (ks, vs, kr, vr, rows) in enumerate(
                ((ksend0, vsend0, krecv0, vrecv0, CHUNK_ROWS[0]),
                 (ksend1, vsend1, krecv1, vrecv1, CHUNK_ROWS[1]))):
            @pl.when(my == s)
            def _(s=s, ks=ks, vs=vs, kr=kr, vr=vr, rows=rows):
                dsts = [(s + 2) % N_DEV, (s + 1) % N_DEV, (s + 3) % N_DEV]
                for cnt, j in enumerate(dsts):
                    ks[cnt] = k_ref[:, :rows, j * HD:(j + 1) * HD].astype(jnp.bfloat16)
                    pltpu.make_async_remote_copy(
                        src_ref=ks.at[cnt], dst_ref=kr,
                        send_sem=ksend_sems.at[cnt], recv_sem=krecv_sems.at[s],
                        device_id=(j,), device_id_type=_DevId.MESH,
                    ).start()
                for cnt, j in enumerate(dsts):
                    vs[cnt] = v_ref[:, :rows, j * HD:(j + 1) * HD].astype(jnp.bfloat16)
                    pltpu.make_async_remote_copy(
                        src_ref=vs.at[cnt], dst_ref=vr,
                        send_sem=vsend_sems.at[cnt], recv_sem=vrecv_sems.at[s],
                        device_id=(j,), device_id_type=_DevId.MESH,
                    ).start()
                kr[:, :, :] = k_ref[:, :rows, s * HD:(s + 1) * HD].astype(jnp.bfloat16)
                vr[:, :, :] = v_ref[:, :rows, s * HD:(s + 1) * HD].astype(jnp.bfloat16)

        Q = []
        for b in range(B):
            Q.append((lax.dot_general(
                x_ref[b].astype(jnp.bfloat16), wq_ref[:, :].astype(jnp.bfloat16),
                (((1,), (0,)), ((), ())), preferred_element_type=jnp.float32)
                * (0.125 * 1.4426950408889634)).astype(jnp.bfloat16))

        def bias(nrow, row0, ncol, col0):
            r = lax.broadcasted_iota(jnp.int32, (nrow, ncol), 0) + row0
            c = lax.broadcasted_iota(jnp.int32, (nrow, ncol), 1) + col0
            return jnp.where(jnp.abs(r - c) <= WINDOW, 0.0, NEG).astype(jnp.float32)

        bias0 = bias(Sq, 0, CHUNK_ROWS[0], 0)
        bias1 = bias(CHUNK_ROWS[1], Sq - CHUNK_ROWS[1], CHUNK_ROWS[1], CHUNK_ROWS[0])

        def wait_one(c, ref, send_sems, recv_sems):
            @pl.when(my != c)
            def _():
                pltpu.make_async_remote_copy(
                    src_ref=ref, dst_ref=ref,
                    send_sem=send_sems.at[0], recv_sem=recv_sems.at[c],
                    device_id=(0,), device_id_type=_DevId.MESH,
                ).wait_recv()

        wait_one(0, krecv0, ksend_sems, krecv_sems)
        e0s = [[None] * H for _ in range(B)]
        l0 = [[None] * H for _ in range(B)]
        for b in range(B):
            for h in range(H):
                q = Q[b][:, h * Dh:(h + 1) * Dh]
                e0 = jnp.exp2(lax.dot_general(
                    q, krecv0[b, :, h * Dh:(h + 1) * Dh], (((1,), (1,)), ((), ())),
                    preferred_element_type=jnp.float32) + bias0)
                l0[b][h] = jnp.sum(e0, axis=1, keepdims=True)
                e0s[b][h] = e0.astype(jnp.bfloat16)

        wait_one(0, vrecv0, vsend_sems, vrecv_sems)
        c0 = [[None] * H for _ in range(B)]
        for b in range(B):
            for h in range(H):
                c0[b][h] = lax.dot_general(
                    e0s[b][h], vrecv0[b, :, h * Dh:(h + 1) * Dh],
                    (((1,), (0,)), ((), ())), preferred_element_type=jnp.float32)

        wait_one(1, krecv1, ksend_sems, krecv_sems)
        SqH = Sq - CHUNK_ROWS[1]
        e1s = [[None] * H for _ in range(B)]
        l1 = [[None] * H for _ in range(B)]
        for b in range(B):
            for h in range(H):
                q_hi = Q[b][SqH:, h * Dh:(h + 1) * Dh]
                e1 = jnp.exp2(lax.dot_general(
                    q_hi, krecv1[b, :, h * Dh:(h + 1) * Dh], (((1,), (1,)), ((), ())),
                    preferred_element_type=jnp.float32) + bias1)
                l1[b][h] = jnp.sum(e1, axis=1, keepdims=True)
                e1s[b][h] = e1.astype(jnp.bfloat16)

        wait_one(1, vrecv1, vsend_sems, vrecv_sems)
        for b in range(B):
            acc = jnp.zeros((Sq, D), jnp.float32)
            for h in range(H):
                ctx_hi = c0[b][h][SqH:, :] + lax.dot_general(
                    e1s[b][h], vrecv1[b, :, h * Dh:(h + 1) * Dh],
                    (((1,), (0,)), ((), ())), preferred_element_type=jnp.float32)
                ctx = jnp.concatenate(
                    [c0[b][h][:SqH, :] * (1.0 / l0[b][h][:SqH, :]),
                     ctx_hi * (1.0 / (l0[b][h][SqH:, :] + l1[b][h]))], axis=0)
                acc = acc + lax.dot_general(
                    ctx.astype(jnp.bfloat16),
                    wo_ref[h * Dh:(h + 1) * Dh, :].astype(jnp.bfloat16),
                    (((1,), (0,)), ((), ())), preferred_element_type=jnp.float32)
            pbuf[b] = acc.astype(jnp.bfloat16)
            for k in (2, 1, 3):
                d = (my + k) % N_DEV
                pltpu.make_async_remote_copy(
                    src_ref=pbuf.at[b, pl.ds(d * SqQ, SqQ), :],
                    dst_ref=rsrecv.at[N_DEV - 1 - k, b],
                    send_sem=rssend_sems.at[k - 1, b],
                    recv_sem=rsrecv_sems.at[N_DEV - 1 - k, b],
                    device_id=(d,), device_id_type=_DevId.MESH,
                ).start()

        for s, (ks, vs, kr, vr) in enumerate(
                ((ksend0, vsend0, krecv0, vrecv0),
                 (ksend1, vsend1, krecv1, vrecv1))):
            @pl.when(my == s)
            def _(s=s, ks=ks, vs=vs, kr=kr, vr=vr):
                for cnt in range(N_DEV - 1):
                    pltpu.make_async_remote_copy(
                        src_ref=ks.at[cnt], dst_ref=kr,
                        send_sem=ksend_sems.at[cnt], recv_sem=krecv_sems.at[s],
                        device_id=(0,), device_id_type=_DevId.MESH,
                    ).wait_send()
                    pltpu.make_async_remote_copy(
                        src_ref=vs.at[cnt], dst_ref=vr,
                        send_sem=vsend_sems.at[cnt], recv_sem=vrecv_sems.at[s],
                        device_id=(0,), device_id_type=_DevId.MESH,
                    ).wait_send()

        sums = []
        for b in range(B):
            for j in range(N_DEV - 1):
                pltpu.make_async_remote_copy(
                    src_ref=rsrecv.at[j, b], dst_ref=rsrecv.at[j, b],
                    send_sem=rssend_sems.at[0, b], recv_sem=rsrecv_sems.at[j, b],
                    device_id=(0,), device_id_type=_DevId.MESH,
                ).wait_recv()
            sum_q = pbuf[b, pl.ds(my * SqQ, SqQ), :].astype(jnp.float32)
            for j in range(N_DEV - 1):
                sum_q = sum_q + rsrecv[j, b].astype(jnp.float32)
            sums.append(sum_q)
            agsend[b] = sum_q.astype(jnp.bfloat16)
            for k in (2, 1, 3):
                pltpu.make_async_remote_copy(
                    src_ref=agsend.at[b], dst_ref=agrecv.at[N_DEV - 1 - k, b],
                    send_sem=agsend_sems.at[k - 1, b],
                    recv_sem=agrecv_sems.at[N_DEV - 1 - k, b],
                    device_id=((my + k) % N_DEV,), device_id_type=_DevId.MESH,
                ).start()

        for b in range(B):
            out_ref[b, pl.ds(my * SqQ, SqQ), :] = sums[b]
            for j in range(N_DEV - 1):
                pltpu.make_async_remote_copy(
                    src_ref=agsend.at[b], dst_ref=agrecv.at[j, b],
                    send_sem=agsend_sems.at[0, b], recv_sem=agrecv_sems.at[j, b],
                    device_id=(0,), device_id_type=_DevId.MESH,
                ).wait_recv()
                src = (my + j + 1) % N_DEV
                out_ref[b, pl.ds(src * SqQ, SqQ), :] = \
                    agrecv[j, b].astype(jnp.float32)

        for b in range(B):
            for k in range(1, N_DEV):
                pltpu.make_async_remote_copy(
                    src_ref=pbuf.at[b, pl.ds(0, SqQ), :], dst_ref=rsrecv.at[0, b],
                    send_sem=rssend_sems.at[k - 1, b], recv_sem=rsrecv_sems.at[0, b],
                    device_id=(0,), device_id_type=_DevId.MESH,
                ).wait_send()
                pltpu.make_async_remote_copy(
                    src_ref=agsend.at[b], dst_ref=agrecv.at[0, b],
                    send_sem=agsend_sems.at[k - 1, b], recv_sem=agrecv_sems.at[0, b],
                    device_id=(0,), device_id_type=_DevId.MESH,
                ).wait_send()

    return pl.pallas_call(
        body,
        out_shape=jax.ShapeDtypeStruct((B, Sq, D), jnp.float32),
        in_specs=[pl.BlockSpec(memory_space=pltpu.VMEM)] * 5,
        out_specs=pl.BlockSpec(memory_space=pltpu.VMEM),
        scratch_shapes=[
            pltpu.VMEM((N_DEV - 1, B, CHUNK_ROWS[0], HD), jnp.bfloat16),
            pltpu.VMEM((N_DEV - 1, B, CHUNK_ROWS[0], HD), jnp.bfloat16),
            pltpu.VMEM((N_DEV - 1, B, CHUNK_ROWS[1], HD), jnp.bfloat16),
            pltpu.VMEM((N_DEV - 1, B, CHUNK_ROWS[1], HD), jnp.bfloat16),
            pltpu.VMEM((B, CHUNK_ROWS[0], HD), jnp.bfloat16),
            pltpu.VMEM((B, CHUNK_ROWS[0], HD), jnp.bfloat16),
            pltpu.VMEM((B, CHUNK_ROWS[1], HD), jnp.bfloat16),
            pltpu.VMEM((B, CHUNK_ROWS[1], HD), jnp.bfloat16),
            pltpu.VMEM((B, Sq, D), jnp.bfloat16),
            pltpu.VMEM((N_DEV - 1, B, Sq // N_DEV, D), jnp.bfloat16),
            pltpu.VMEM((B, Sq // N_DEV, D), jnp.bfloat16),
            pltpu.VMEM((N_DEV - 1, B, Sq // N_DEV, D), jnp.bfloat16),
            pltpu.SemaphoreType.DMA((N_DEV - 1,)),
            pltpu.SemaphoreType.DMA((N_DEV - 1,)),
            pltpu.SemaphoreType.DMA((NCHUNK,)),
            pltpu.SemaphoreType.DMA((NCHUNK,)),
            pltpu.SemaphoreType.DMA((N_DEV - 1, B)),
            pltpu.SemaphoreType.DMA((N_DEV - 1, B)),
            pltpu.SemaphoreType.DMA((N_DEV - 1, B)),
            pltpu.SemaphoreType.DMA((N_DEV - 1, B)),
        ],
        compiler_params=pltpu.CompilerParams(collective_id=0),
    )(x, Wq, K2, V2, Wo)
